# baseline (speedup 1.0000x reference)
import functools

import jax
import jax.numpy as jnp
from jax import lax
from jax.experimental import pallas as pl
from jax.experimental.pallas import tpu as pltpu
from jax.experimental.pallas import tpu_sc as plsc

NUM_RINGS = 50
EMBED_DIM = 64
FLAT = NUM_RINGS * EMBED_DIM
BATCH = 16384
SC_ROWS = 8192

NC = 2
NS = 16
LANES = 16
NW = NC * NS
ROWS_PER_W = SC_ROWS // NW
CH = 16
NSTEPS = ROWS_PER_W // CH
NVREG = FLAT // LANES


def _sc_body(x_hbm, w_hbm, o_hbm, wv, b0, b1, si0, si1, so0, so1):
    cid = lax.axis_index("c")
    sid = lax.axis_index("s")
    wid = sid * NC + cid
    base = wid * ROWS_PER_W

    pltpu.sync_copy(w_hbm, wv)

    bufs = (b0, b1)
    isems = (si0, si1)
    osems = (so0, so1)
    in_h = [None, None]
    out_h = [None, None]

    in_h[0] = pltpu.async_copy(x_hbm.at[pl.ds(base, CH)], bufs[0], isems[0])

    for step in range(NSTEPS):
        k = step % 2
        nk = (step + 1) % 2
        if step + 1 < NSTEPS:
            if step >= 1:
                out_h[nk].wait()
            in_h[nk] = pltpu.async_copy(
                x_hbm.at[pl.ds(base + (step + 1) * CH, CH)], bufs[nk], isems[nk])
        in_h[k].wait()

        buf = bufs[k]

        def jbody(j, _, buf=buf):
            w16 = wv[pl.ds(j * LANES, LANES)]
            for cc in range(CH):
                buf[cc, pl.ds(j * LANES, LANES)] = (
                    buf[cc, pl.ds(j * LANES, LANES)] + w16)
            return 0

        lax.fori_loop(0, NVREG, jbody, 0)

        out_h[k] = pltpu.async_copy(
            buf, o_hbm.at[pl.ds(base + step * CH, CH)], osems[k])

    out_h[0].wait()
    out_h[1].wait()


def _sc_call(xf, wf):
    mesh = plsc.VectorSubcoreMesh(core_axis_name="c", subcore_axis_name="s")
    return pl.kernel(
        _sc_body,
        out_type=jax.ShapeDtypeStruct((SC_ROWS, FLAT), jnp.float32),
        mesh=mesh,
        scratch_types=[
            pltpu.VMEM((FLAT,), jnp.float32),
            pltpu.VMEM((CH, FLAT), jnp.float32),
            pltpu.VMEM((CH, FLAT), jnp.float32),
            pltpu.SemaphoreType.DMA,
            pltpu.SemaphoreType.DMA,
            pltpu.SemaphoreType.DMA,
            pltpu.SemaphoreType.DMA,
        ],
    )(xf, wf)


def _tc_body(x_ref, w_ref, o_ref):
    o_ref[...] = x_ref[...] + w_ref[...]


def _tc_call(xf, wf2):
    n = xf.shape[0]
    bm = 1024
    return pl.pallas_call(
        _tc_body,
        grid=(n // bm,),
        in_specs=[
            pl.BlockSpec((bm, FLAT), lambda i: (i, 0)),
            pl.BlockSpec((1, FLAT), lambda i: (0, 0)),
        ],
        out_specs=pl.BlockSpec((bm, FLAT), lambda i: (i, 0)),
        out_shape=jax.ShapeDtypeStruct((n, FLAT), jnp.float32),
    )(xf, wf2)


def kernel(x, W):
    B = x.shape[0]
    xf = x.reshape(B, FLAT)
    wf = W.reshape(FLAT)
    o1 = _sc_call(xf[:SC_ROWS], wf)
    o2 = _tc_call(xf[SC_ROWS:], wf.reshape(1, FLAT))
    return (o1, o2)
